# Initial kernel scaffold; baseline (speedup 1.0000x reference)
import jax
import jax.numpy as jnp
from jax.experimental import pallas as pl


def kernel(x, w1, b1, w2, b2):
    raise NotImplementedError("write your pallas kernel here")


# speedup vs baseline: 1.1642x; 1.1642x over previous
import jax
import jax.numpy as jnp
from jax.experimental import pallas as pl
from jax.experimental.pallas import tpu as pltpu

_PACK = 8
_TM = 1024


def _mlp_packed_kernel(x_ref, w1_ref, b1_ref, w2_ref, b2_ref, out_ref):
    x = x_ref[...]
    h = jnp.dot(x.astype(jnp.bfloat16), w1_ref[...],
                preferred_element_type=jnp.float32) + b1_ref[...]
    h = jnp.maximum(h, 0.0)
    y = jnp.dot(h.astype(jnp.bfloat16), w2_ref[...],
                preferred_element_type=jnp.float32) + b2_ref[...]
    out_ref[...] = y.astype(out_ref.dtype)


def kernel(x, w1, b1, w2, b2):
    M, f_in = x.shape
    hidden = w1.shape[1]
    f_out = w2.shape[1]

    k = _PACK
    pad_rows = (-M) % k
    if pad_rows:
        x = jnp.concatenate([x, jnp.zeros((pad_rows, f_in), x.dtype)], axis=0)
    Mp = (M + pad_rows) // k
    fi, hi, fo = f_in * k, hidden * k, f_out * k

    xp = x.reshape(Mp, fi)

    w1b = jax.scipy.linalg.block_diag(*([w1] * k)).astype(jnp.bfloat16)
    w2b = jax.scipy.linalg.block_diag(*([w2] * k)).astype(jnp.bfloat16)
    b1b = jnp.tile(b1, (1, k))
    b2b = jnp.tile(b2, (1, k))

    tm = min(_TM, Mp)
    grid = (pl.cdiv(Mp, tm),)

    yp = pl.pallas_call(
        _mlp_packed_kernel,
        out_shape=jax.ShapeDtypeStruct((Mp, fo), x.dtype),
        grid=grid,
        in_specs=[
            pl.BlockSpec((tm, fi), lambda i: (i, 0)),
            pl.BlockSpec((fi, hi), lambda i: (0, 0)),
            pl.BlockSpec((1, hi), lambda i: (0, 0)),
            pl.BlockSpec((hi, fo), lambda i: (0, 0)),
            pl.BlockSpec((1, fo), lambda i: (0, 0)),
        ],
        out_specs=pl.BlockSpec((tm, fo), lambda i: (i, 0)),
        compiler_params=pltpu.CompilerParams(
            dimension_semantics=("parallel",)),
    )(xp, w1b, b1b, w2b, b2b)

    return yp.reshape(Mp * k, f_out)[:M]


# speedup vs baseline: 1.2846x; 1.1034x over previous
import jax
import jax.numpy as jnp
from jax.experimental import pallas as pl
from jax.experimental.pallas import tpu as pltpu

_TM = 8192


def _mlp_kernel(x_ref, w1_ref, b1_ref, w2_ref, b2_ref, out_ref):
    x = x_ref[...]
    h = jnp.dot(x.astype(jnp.bfloat16), w1_ref[...].astype(jnp.bfloat16),
                preferred_element_type=jnp.float32) + b1_ref[...]
    h = jnp.maximum(h, 0.0)
    y = jnp.dot(h.astype(jnp.bfloat16), w2_ref[...].astype(jnp.bfloat16),
                preferred_element_type=jnp.float32) + b2_ref[...]
    out_ref[...] = y.astype(out_ref.dtype)


def kernel(x, w1, b1, w2, b2):
    M, f_in = x.shape
    hidden = w1.shape[1]
    f_out = w2.shape[1]

    tm = min(_TM, M)
    grid = (pl.cdiv(M, tm),)

    return pl.pallas_call(
        _mlp_kernel,
        out_shape=jax.ShapeDtypeStruct((M, f_out), x.dtype),
        grid=grid,
        in_specs=[
            pl.BlockSpec((tm, f_in), lambda i: (i, 0)),
            pl.BlockSpec((f_in, hidden), lambda i: (0, 0)),
            pl.BlockSpec((1, hidden), lambda i: (0, 0)),
            pl.BlockSpec((hidden, f_out), lambda i: (0, 0)),
            pl.BlockSpec((1, f_out), lambda i: (0, 0)),
        ],
        out_specs=pl.BlockSpec((tm, f_out), lambda i: (i, 0)),
        compiler_params=pltpu.CompilerParams(
            dimension_semantics=("parallel",)),
    )(x, w1, b1, w2, b2)


# speedup vs baseline: 2.3070x; 1.7959x over previous
import jax
import jax.numpy as jnp
from jax.experimental import pallas as pl
from jax.experimental.pallas import tpu as pltpu

_TM = 8192


def _mlp_t_kernel(x_ref, w1_ref, b1t_ref, w2_ref, b2t_ref, out_ref):
    x = x_ref[...].astype(jnp.bfloat16)
    ht = jax.lax.dot_general(
        w1_ref[...], x, (((0,), (1,)), ((), ())),
        preferred_element_type=jnp.float32)
    ht = jnp.maximum(ht + b1t_ref[...], 0.0).astype(jnp.bfloat16)
    yt = jax.lax.dot_general(
        w2_ref[...], ht, (((0,), (0,)), ((), ())),
        preferred_element_type=jnp.float32)
    out_ref[...] = (yt + b2t_ref[...]).astype(out_ref.dtype)


def kernel(x, w1, b1, w2, b2):
    M, f_in = x.shape
    hidden = w1.shape[1]
    f_out = w2.shape[1]

    w1c = w1.astype(jnp.bfloat16)
    w2c = w2.astype(jnp.bfloat16)
    b1t = b1.reshape(hidden, 1)
    b2t = b2.reshape(f_out, 1)

    tm = min(_TM, M)
    grid = (pl.cdiv(M, tm),)

    yt = pl.pallas_call(
        _mlp_t_kernel,
        out_shape=jax.ShapeDtypeStruct((f_out, M), x.dtype),
        grid=grid,
        in_specs=[
            pl.BlockSpec((tm, f_in), lambda i: (i, 0)),
            pl.BlockSpec((f_in, hidden), lambda i: (0, 0)),
            pl.BlockSpec((hidden, 1), lambda i: (0, 0)),
            pl.BlockSpec((hidden, f_out), lambda i: (0, 0)),
            pl.BlockSpec((f_out, 1), lambda i: (0, 0)),
        ],
        out_specs=pl.BlockSpec((f_out, tm), lambda i: (0, i)),
        compiler_params=pltpu.CompilerParams(
            dimension_semantics=("parallel",)),
    )(x, w1c, b1t, w2c, b2t)

    return yt.T


# speedup vs baseline: 2.5059x; 1.0862x over previous
import jax
import jax.numpy as jnp
from jax.experimental import pallas as pl
from jax.experimental.pallas import tpu as pltpu

_TM = 16384


def _mlp_t_kernel(x_ref, w1_ref, b1t_ref, w2_ref, b2t_ref, out_ref):
    x = x_ref[...].astype(jnp.bfloat16)
    ht = jax.lax.dot_general(
        w1_ref[...], x, (((0,), (1,)), ((), ())),
        preferred_element_type=jnp.float32)
    ht = jnp.maximum(ht + b1t_ref[...], 0.0).astype(jnp.bfloat16)
    yt = jax.lax.dot_general(
        w2_ref[...], ht, (((0,), (0,)), ((), ())),
        preferred_element_type=jnp.float32)
    out_ref[...] = (yt + b2t_ref[...]).astype(out_ref.dtype)


def kernel(x, w1, b1, w2, b2):
    M, f_in = x.shape
    hidden = w1.shape[1]
    f_out = w2.shape[1]

    w1c = w1.astype(jnp.bfloat16)
    w2c = w2.astype(jnp.bfloat16)
    b1t = b1.reshape(hidden, 1)
    b2t = b2.reshape(f_out, 1)

    tm = min(_TM, M)
    grid = (pl.cdiv(M, tm),)

    yt = pl.pallas_call(
        _mlp_t_kernel,
        out_shape=jax.ShapeDtypeStruct((f_out, M), x.dtype),
        grid=grid,
        in_specs=[
            pl.BlockSpec((tm, f_in), lambda i: (i, 0)),
            pl.BlockSpec((f_in, hidden), lambda i: (0, 0)),
            pl.BlockSpec((hidden, 1), lambda i: (0, 0)),
            pl.BlockSpec((hidden, f_out), lambda i: (0, 0)),
            pl.BlockSpec((f_out, 1), lambda i: (0, 0)),
        ],
        out_specs=pl.BlockSpec((f_out, tm), lambda i: (0, i)),
        compiler_params=pltpu.CompilerParams(
            dimension_semantics=("parallel",)),
    )(x, w1c, b1t, w2c, b2t)

    return yt.T


# speedup vs baseline: 2.5245x; 1.0074x over previous
import jax
import jax.numpy as jnp
from jax.experimental import pallas as pl
from jax.experimental.pallas import tpu as pltpu

_TM = 32768


def _mlp_t_kernel(x_ref, w1_ref, b1t_ref, w2_ref, b2t_ref, out_ref):
    x = x_ref[...].astype(jnp.bfloat16)
    ht = jax.lax.dot_general(
        w1_ref[...], x, (((0,), (1,)), ((), ())),
        preferred_element_type=jnp.float32)
    ht = jnp.maximum(ht + b1t_ref[...], 0.0).astype(jnp.bfloat16)
    yt = jax.lax.dot_general(
        w2_ref[...], ht, (((0,), (0,)), ((), ())),
        preferred_element_type=jnp.float32)
    out_ref[...] = (yt + b2t_ref[...]).astype(out_ref.dtype)


def kernel(x, w1, b1, w2, b2):
    M, f_in = x.shape
    hidden = w1.shape[1]
    f_out = w2.shape[1]

    w1c = w1.astype(jnp.bfloat16)
    w2c = w2.astype(jnp.bfloat16)
    b1t = b1.reshape(hidden, 1)
    b2t = b2.reshape(f_out, 1)

    tm = min(_TM, M)
    grid = (pl.cdiv(M, tm),)

    yt = pl.pallas_call(
        _mlp_t_kernel,
        out_shape=jax.ShapeDtypeStruct((f_out, M), x.dtype),
        grid=grid,
        in_specs=[
            pl.BlockSpec((tm, f_in), lambda i: (i, 0)),
            pl.BlockSpec((f_in, hidden), lambda i: (0, 0)),
            pl.BlockSpec((hidden, 1), lambda i: (0, 0)),
            pl.BlockSpec((hidden, f_out), lambda i: (0, 0)),
            pl.BlockSpec((f_out, 1), lambda i: (0, 0)),
        ],
        out_specs=pl.BlockSpec((f_out, tm), lambda i: (0, i)),
        compiler_params=pltpu.CompilerParams(
            dimension_semantics=("parallel",)),
    )(x, w1c, b1t, w2c, b2t)

    return yt.T
